# baseline (speedup 1.0000x reference)
import jax
import jax.numpy as jnp
from jax.experimental import pallas as pl
from jax.experimental.pallas import tpu as pltpu

_N = 10000
_D = 128
_BM = 200


def _seq_body(x_ref, wt_ref, seq_ref):
    x = x_ref[...].astype(jnp.bfloat16)
    wt = wt_ref[...].astype(jnp.bfloat16)
    seq_ref[...] = jnp.dot(x, wt, preferred_element_type=jnp.float32).astype(
        jnp.bfloat16
    )


def _spmm_body(adj_ref, seq_ref, bias_ref, a_ref, out_ref):
    a = adj_ref[...].astype(jnp.bfloat16)
    acc = jnp.dot(a, seq_ref[...], preferred_element_type=jnp.float32)
    z = acc + bias_ref[0:1, :]
    slope = a_ref[0, 0]
    z = jnp.where(z >= 0.0, z, slope * z)
    out_ref[...] = jnp.maximum(z, 0.0)


def kernel(x, adj, W, bias, prelu_a):
    n, d_in = x.shape
    d_out = W.shape[0]

    seq_fts = pl.pallas_call(
        _seq_body,
        out_shape=jax.ShapeDtypeStruct((n, d_out), jnp.bfloat16),
        in_specs=[
            pl.BlockSpec((n, d_in), lambda: (0, 0)),
            pl.BlockSpec((d_in, d_out), lambda: (0, 0)),
        ],
        out_specs=pl.BlockSpec((n, d_out), lambda: (0, 0)),
    )(x, W.T)

    bias2 = jnp.broadcast_to(bias.reshape(1, d_out), (8, d_out))
    a2 = prelu_a.reshape(1, 1)

    out = pl.pallas_call(
        _spmm_body,
        grid=(n // _BM,),
        out_shape=jax.ShapeDtypeStruct((n, d_out), jnp.float32),
        in_specs=[
            pl.BlockSpec((_BM, n), lambda i: (i, 0)),
            pl.BlockSpec((n, d_out), lambda i: (0, 0)),
            pl.BlockSpec((8, d_out), lambda i: (0, 0)),
            pl.BlockSpec(memory_space=pltpu.SMEM),
        ],
        out_specs=pl.BlockSpec((_BM, d_out), lambda i: (i, 0)),
        compiler_params=pltpu.CompilerParams(
            dimension_semantics=("parallel",),
        ),
    )(adj, seq_fts, bias2, a2)

    return out


# speedup vs baseline: 1.0259x; 1.0259x over previous
import jax
import jax.numpy as jnp
from jax.experimental import pallas as pl
from jax.experimental.pallas import tpu as pltpu

_BM = 200


def _body(x_ref, wt_ref, adj_ref, bias_ref, a_ref, out_ref, seq_ref):
    @pl.when(pl.program_id(0) == 0)
    def _():
        xb = x_ref[...].astype(jnp.bfloat16)
        wb = wt_ref[...].astype(jnp.bfloat16)
        seq_ref[...] = jnp.dot(
            xb, wb, preferred_element_type=jnp.float32
        ).astype(jnp.bfloat16)

    a = adj_ref[...].astype(jnp.bfloat16)
    acc = jnp.dot(a, seq_ref[...], preferred_element_type=jnp.float32)
    z = acc + bias_ref[0:1, :]
    slope = a_ref[0, 0]
    z = jnp.where(z >= 0.0, z, slope * z)
    out_ref[...] = jnp.maximum(z, 0.0)


def kernel(x, adj, W, bias, prelu_a):
    n, d_in = x.shape
    d_out = W.shape[0]

    bias2 = jnp.broadcast_to(bias.reshape(1, d_out), (8, d_out))
    a2 = prelu_a.reshape(1, 1)

    out = pl.pallas_call(
        _body,
        grid=(n // _BM,),
        out_shape=jax.ShapeDtypeStruct((n, d_out), jnp.float32),
        in_specs=[
            pl.BlockSpec((n, d_in), lambda i: (0, 0)),
            pl.BlockSpec((d_in, d_out), lambda i: (0, 0)),
            pl.BlockSpec((_BM, n), lambda i: (i, 0)),
            pl.BlockSpec((8, d_out), lambda i: (0, 0)),
            pl.BlockSpec(memory_space=pltpu.SMEM),
        ],
        out_specs=pl.BlockSpec((_BM, d_out), lambda i: (i, 0)),
        scratch_shapes=[pltpu.VMEM((n, d_out), jnp.bfloat16)],
        compiler_params=pltpu.CompilerParams(
            dimension_semantics=("arbitrary",),
        ),
    )(x, W.T, adj, bias2, a2)

    return out


# speedup vs baseline: 1.0385x; 1.0123x over previous
import jax
import jax.numpy as jnp
from jax.experimental import pallas as pl
from jax.experimental.pallas import tpu as pltpu

_BM = 400


def _body(x_ref, wt_ref, adj_ref, bias_ref, a_ref, out_ref, seq_ref):
    @pl.when(pl.program_id(0) == 0)
    def _():
        xb = x_ref[...].astype(jnp.bfloat16)
        wb = wt_ref[...].astype(jnp.bfloat16)
        seq_ref[...] = jnp.dot(
            xb, wb, preferred_element_type=jnp.float32
        ).astype(jnp.bfloat16)

    a = adj_ref[...].astype(jnp.bfloat16)
    acc = jnp.dot(a, seq_ref[...], preferred_element_type=jnp.float32)
    z = acc + bias_ref[0:1, :]
    slope = a_ref[0, 0]
    z = jnp.where(z >= 0.0, z, slope * z)
    out_ref[...] = jnp.maximum(z, 0.0)


def kernel(x, adj, W, bias, prelu_a):
    n, d_in = x.shape
    d_out = W.shape[0]

    bias2 = jnp.broadcast_to(bias.reshape(1, d_out), (8, d_out))
    a2 = prelu_a.reshape(1, 1)

    out = pl.pallas_call(
        _body,
        grid=(n // _BM,),
        out_shape=jax.ShapeDtypeStruct((n, d_out), jnp.float32),
        in_specs=[
            pl.BlockSpec((n, d_in), lambda i: (0, 0)),
            pl.BlockSpec((d_in, d_out), lambda i: (0, 0)),
            pl.BlockSpec((_BM, n), lambda i: (i, 0)),
            pl.BlockSpec((8, d_out), lambda i: (0, 0)),
            pl.BlockSpec(memory_space=pltpu.SMEM),
        ],
        out_specs=pl.BlockSpec((_BM, d_out), lambda i: (i, 0)),
        scratch_shapes=[pltpu.VMEM((n, d_out), jnp.bfloat16)],
        compiler_params=pltpu.CompilerParams(
            dimension_semantics=("arbitrary",),
        ),
    )(x, W.T, adj, bias2, a2)

    return out
